# Initial kernel scaffold; baseline (speedup 1.0000x reference)
import jax
import jax.numpy as jnp
from jax.experimental import pallas as pl


def kernel(given4rank1_stimulus_set, percept_table):
    raise NotImplementedError("write your pallas kernel here")


# speedup vs baseline: 7.9520x; 7.9520x over previous
import functools

import jax
import jax.numpy as jnp
from jax import lax
from jax.experimental import pallas as pl
from jax.experimental.pallas import tpu as pltpu
from jax.experimental.pallas import tpu_sc as plsc

B = 16384
NC, NS, L = 2, 16, 16
NW = NC * NS
BPW = B // NW
CHUNKS = BPW // L
TPAD = 32

_MAGIC = 0x5F3759DF


def _rsqrt(x):
    i = plsc.bitcast(x, jnp.int32)
    i = jnp.int32(_MAGIC) - lax.shift_right_logical(i, 1)
    y = plsc.bitcast(i, jnp.float32)
    half = jnp.float32(0.5) * x
    for _ in range(3):
        y = y * (jnp.float32(1.5) - half * y * y)
    return y


def _sc_body(idx_hbm, tab_hbm, out_hbm, idx_v, tab_v, out_v):
    wid = lax.axis_index("s") * NC + lax.axis_index("c")
    base = wid * BPW
    pltpu.sync_copy(idx_hbm.at[pl.ds(base * 5, BPW * 5)], idx_v)
    pltpu.sync_copy(tab_hbm, tab_v)

    lane = lax.iota(jnp.int32, L)
    lane5 = lane * 5
    lane4 = lane * 4

    def chunk(i, _):
        off5 = i * (L * 5)
        off4 = i * (L * 4)
        pos = off5 + lane5
        iq = plsc.load_gather(idx_v, [pos])
        zq = [plsc.load_gather(tab_v, [iq + d * TPAD]) for d in range(3)]
        s_all = []
        denom = jnp.zeros((L,), jnp.float32)
        ir_all = []
        for j in range(1, 5):
            ir = plsc.load_gather(idx_v, [pos + j])
            ir_all.append(ir)
            d2 = jnp.zeros((L,), jnp.float32)
            for d in range(3):
                zr = plsc.load_gather(tab_v, [ir + d * TPAD])
                diff = zq[d] - zr
                d2 = d2 + diff * diff
            dist = d2 * _rsqrt(d2)
            s = jnp.exp(jnp.float32(-10.0) * dist)
            s = jnp.where(ir != 0, s, jnp.float32(0.0))
            s_all.append(s)
            denom = denom + s
        inv = jnp.float32(1.0) / jnp.maximum(denom, jnp.float32(1e-12))
        for j in range(4):
            plsc.store_scatter(out_v, [off4 + lane4 + j], s_all[j] * inv)
        return 0

    lax.fori_loop(0, CHUNKS, chunk, 0)
    pltpu.sync_copy(out_v, out_hbm.at[pl.ds(base * 4, BPW * 4)])


@functools.partial(jax.jit, static_argnames=())
def kernel(given4rank1_stimulus_set, percept_table):
    idx_flat = given4rank1_stimulus_set.astype(jnp.int32).reshape(-1)
    tab = jnp.pad(percept_table.astype(jnp.float32).T, ((0, 0), (0, TPAD - percept_table.shape[0])))
    tab_flat = tab.reshape(-1)

    mesh = plsc.VectorSubcoreMesh(
        core_axis_name="c", subcore_axis_name="s", num_cores=NC, num_subcores=NS
    )
    out = pl.kernel(
        _sc_body,
        out_type=jax.ShapeDtypeStruct((B * 4,), jnp.float32),
        mesh=mesh,
        scratch_types=[
            pltpu.VMEM((BPW * 5,), jnp.int32),
            pltpu.VMEM((3 * TPAD,), jnp.float32),
            pltpu.VMEM((BPW * 4,), jnp.float32),
        ],
        compiler_params=pltpu.CompilerParams(needs_layout_passes=False),
    )(idx_flat, tab_flat)
    return out.reshape(B, 4)


# speedup vs baseline: 7.9579x; 1.0007x over previous
import functools

import jax
import jax.numpy as jnp
from jax import lax
from jax.experimental import pallas as pl
from jax.experimental.pallas import tpu as pltpu
from jax.experimental.pallas import tpu_sc as plsc

B = 16384
NC, NS, L = 2, 16, 16
NW = NC * NS
BPW = B // NW
CHUNKS = BPW // L
TPAD = 32
NROWS = 21

_MAGIC = 0x5F3759DF


def _rsqrt(x):
    i = plsc.bitcast(x, jnp.int32)
    i = jnp.int32(_MAGIC) - lax.shift_right_logical(i, 1)
    y = plsc.bitcast(i, jnp.float32)
    half = jnp.float32(0.5) * x
    for _ in range(3):
        y = y * (jnp.float32(1.5) - half * y * y)
    return y


def _sc_body(idx_hbm, tab_hbm, out_hbm, idx_v, tab_v, sim_v, out_v, sem):
    wid = lax.axis_index("s") * NC + lax.axis_index("c")
    base = wid * BPW
    pltpu.sync_copy(tab_hbm, tab_v)
    idx_cp = pltpu.async_copy(idx_hbm.at[pl.ds(base * 5, BPW * 5)], idx_v, sem)

    lane = lax.iota(jnp.int32, L)
    lane5 = lane * 5
    lane4 = lane * 4
    mask0 = jnp.where(lane != 0, jnp.float32(1.0), jnp.float32(0.0))

    def pre(q, _):
        qv = jnp.zeros((L,), jnp.int32) + q
        zq = [plsc.load_gather(tab_v, [qv + d * TPAD]) for d in range(3)]
        for half in range(2):
            off = half * L
            d2 = jnp.zeros((L,), jnp.float32)
            for d in range(3):
                zr = tab_v[pl.ds(d * TPAD + off, L)]
                diff = zq[d] - zr
                d2 = d2 + diff * diff
            dist = d2 * _rsqrt(d2)
            s = jnp.exp(jnp.float32(-10.0) * dist)
            if half == 0:
                s = s * mask0
            sim_v[pl.ds(q * TPAD + off, L)] = s
        return 0

    lax.fori_loop(0, NROWS, pre, 0)
    idx_cp.wait()

    def chunk(i, _):
        off5 = i * (L * 5)
        off4 = i * (L * 4)
        pos = off5 + lane5
        iq = plsc.load_gather(idx_v, [pos])
        qb = iq * TPAD
        s_all = []
        denom = jnp.zeros((L,), jnp.float32)
        for j in range(1, 5):
            ir = plsc.load_gather(idx_v, [pos + j])
            s = plsc.load_gather(sim_v, [qb + ir])
            s_all.append(s)
            denom = denom + s
        inv = jnp.float32(1.0) / jnp.maximum(denom, jnp.float32(1e-12))
        for j in range(4):
            plsc.store_scatter(out_v, [off4 + lane4 + j], s_all[j] * inv)
        return 0

    lax.fori_loop(0, CHUNKS, chunk, 0)
    pltpu.sync_copy(out_v, out_hbm.at[pl.ds(base * 4, BPW * 4)])


@functools.partial(jax.jit, static_argnames=())
def kernel(given4rank1_stimulus_set, percept_table):
    idx_flat = given4rank1_stimulus_set.astype(jnp.int32).reshape(-1)
    tab = jnp.pad(percept_table.astype(jnp.float32).T, ((0, 0), (0, TPAD - percept_table.shape[0])))
    tab_flat = tab.reshape(-1)

    mesh = plsc.VectorSubcoreMesh(
        core_axis_name="c", subcore_axis_name="s", num_cores=NC, num_subcores=NS
    )
    out = pl.kernel(
        _sc_body,
        out_type=jax.ShapeDtypeStruct((B * 4,), jnp.float32),
        mesh=mesh,
        scratch_types=[
            pltpu.VMEM((BPW * 5,), jnp.int32),
            pltpu.VMEM((3 * TPAD,), jnp.float32),
            pltpu.VMEM((TPAD * TPAD,), jnp.float32),
            pltpu.VMEM((BPW * 4,), jnp.float32),
            pltpu.SemaphoreType.DMA,
        ],
        compiler_params=pltpu.CompilerParams(needs_layout_passes=False),
    )(idx_flat, tab_flat)
    return out.reshape(B, 4)


# speedup vs baseline: 18.5534x; 2.3314x over previous
import functools

import jax
import jax.numpy as jnp
from jax import lax
from jax.experimental import pallas as pl
from jax.experimental.pallas import tpu as pltpu
from jax.experimental.pallas import tpu_sc as plsc

B = 16384
NC, NS, L = 1, 16, 16
NW = NC * NS
BPW = B // NW
CHUNKS = BPW // L
TPAD = 32
NROWS = 21

_MAGIC = 0x5F3759DF


def _rsqrt(x):
    i = plsc.bitcast(x, jnp.int32)
    i = jnp.int32(_MAGIC) - lax.shift_right_logical(i, 1)
    y = plsc.bitcast(i, jnp.float32)
    half = jnp.float32(0.5) * x
    for _ in range(3):
        y = y * (jnp.float32(1.5) - half * y * y)
    return y


def _sc_body(idx_hbm, tab_hbm, out_hbm, idx_v, tab_v, sim_v, out_v, sem):
    wid = lax.axis_index("s") * NC + lax.axis_index("c")
    base = wid * BPW
    idx_cp = pltpu.async_copy(idx_hbm.at[:, pl.ds(base, BPW)], idx_v, sem)
    pltpu.sync_copy(tab_hbm, tab_v.at[pl.ds(0, 3 * NROWS)])

    lane = lax.iota(jnp.int32, L)
    mask0 = jnp.where(lane != 0, jnp.float32(1.0), jnp.float32(0.0))

    def pre(q, _):
        qv = jnp.zeros((L,), jnp.int32) + q
        zq = [plsc.load_gather(tab_v, [qv + d * NROWS]) for d in range(3)]
        for half in range(2):
            off = half * L
            d2 = jnp.zeros((L,), jnp.float32)
            for d in range(3):
                zr = tab_v[pl.ds(d * NROWS + off, L)]
                diff = zq[d] - zr
                d2 = d2 + diff * diff
            dist = d2 * _rsqrt(d2)
            s = jnp.exp(jnp.float32(-10.0) * dist)
            if half == 0:
                s = s * mask0
            sim_v[pl.ds(q * TPAD + off, L)] = s
        return 0

    lax.fori_loop(0, NROWS, pre, 0)
    idx_cp.wait()

    def chunk(i, _):
        off = i * L
        iq = idx_v[0, pl.ds(off, L)]
        qb = iq * TPAD
        s_all = []
        denom = jnp.zeros((L,), jnp.float32)
        for j in range(1, 5):
            ir = idx_v[j, pl.ds(off, L)]
            s = plsc.load_gather(sim_v, [qb + ir])
            s_all.append(s)
            denom = denom + s
        inv = jnp.float32(1.0) / jnp.maximum(denom, jnp.float32(1e-12))
        for j in range(4):
            out_v[j, pl.ds(off, L)] = s_all[j] * inv
        return 0

    lax.fori_loop(0, CHUNKS, chunk, 0)
    pltpu.sync_copy(out_v, out_hbm.at[:, pl.ds(base, BPW)])


@functools.partial(jax.jit, static_argnames=())
def kernel(given4rank1_stimulus_set, percept_table):
    idx_cm = given4rank1_stimulus_set.astype(jnp.int32).T
    tab_cm = percept_table.astype(jnp.float32).T.reshape(-1)

    mesh = plsc.VectorSubcoreMesh(
        core_axis_name="c", subcore_axis_name="s", num_cores=NC, num_subcores=NS
    )
    out = pl.kernel(
        _sc_body,
        out_type=jax.ShapeDtypeStruct((4, B), jnp.float32),
        mesh=mesh,
        scratch_types=[
            pltpu.VMEM((5, BPW), jnp.int32),
            pltpu.VMEM((80,), jnp.float32),
            pltpu.VMEM((TPAD * TPAD,), jnp.float32),
            pltpu.VMEM((4, BPW), jnp.float32),
            pltpu.SemaphoreType.DMA,
        ],
        compiler_params=pltpu.CompilerParams(needs_layout_passes=False),
    )(idx_cm, tab_cm)
    return out.T
